# baseline (speedup 1.0000x reference)
import functools

import jax
import jax.numpy as jnp
from jax import lax
from jax.experimental import pallas as pl
from jax.experimental.pallas import tpu as pltpu
from jax.experimental.pallas import tpu_sc as plsc

NUM_CORES = 2
NUM_SUBCORES = 16
NW = NUM_CORES * NUM_SUBCORES
CHUNK = 128
KK = 4


@functools.cache
def _build(B, D):
    chunks_pw = B // (NW * CHUNK)
    nstreams = chunks_pw // KK
    mesh = plsc.VectorSubcoreMesh(core_axis_name="c", subcore_axis_name="s")

    @functools.partial(
        pl.kernel,
        out_type=jax.ShapeDtypeStruct((B, D), jnp.float32),
        mesh=mesh,
        scratch_types=[
            pltpu.VMEM((chunks_pw * CHUNK,), jnp.int32),
            pltpu.VMEM((KK * CHUNK, D), jnp.float32),
            pltpu.SemaphoreType.DMA,
        ],
        compiler_params=pltpu.CompilerParams(use_tc_tiling_on_sc=False),
    )
    def _gather(idx_hbm, table_hbm, out_hbm, idx_v, rows_v, sem):
        wid = lax.axis_index("s") * NUM_CORES + lax.axis_index("c")
        c0 = wid * chunks_pw
        pltpu.sync_copy(idx_hbm.at[pl.ds(c0 * CHUNK, chunks_pw * CHUNK)], idx_v)

        @pl.loop(0, nstreams)
        def _s(i):
            pltpu.async_copy(
                table_hbm.at[idx_v.at[pl.ds(i * KK * CHUNK, KK * CHUNK)]],
                rows_v,
                sem,
            ).wait()
            pltpu.sync_copy(
                rows_v, out_hbm.at[pl.ds((c0 + i * KK) * CHUNK, KK * CHUNK)]
            )

    return _gather


@jax.jit
def kernel(token_ids, weights):
    S, T = token_ids.shape
    B = S * T
    D = weights.shape[1]
    idx_flat = token_ids.reshape(B).astype(jnp.int32)
    out = _build(B, D)(idx_flat, weights)
    return out.reshape(S, T, D)


# speedup vs baseline: 1.0076x; 1.0076x over previous
import functools

import jax
import jax.numpy as jnp
from jax import lax
from jax.experimental import pallas as pl
from jax.experimental.pallas import tpu as pltpu
from jax.experimental.pallas import tpu_sc as plsc

NUM_CORES = 2
NUM_SUBCORES = 16
NW = NUM_CORES * NUM_SUBCORES
CHUNK = 128
KK = 4
GROUP = KK * CHUNK


@functools.cache
def _build(B, D):
    chunks_pw = B // (NW * CHUNK)
    ngroups = chunks_pw // KK
    assert ngroups >= 2 and ngroups % 2 == 0
    mesh = plsc.VectorSubcoreMesh(core_axis_name="c", subcore_axis_name="s")

    @functools.partial(
        pl.kernel,
        out_type=jax.ShapeDtypeStruct((B, D), jnp.float32),
        mesh=mesh,
        scratch_types=[
            pltpu.VMEM((chunks_pw * CHUNK,), jnp.int32),
            pltpu.VMEM((GROUP, D), jnp.float32),
            pltpu.VMEM((GROUP, D), jnp.float32),
            pltpu.SemaphoreType.DMA,
            pltpu.SemaphoreType.DMA,
            pltpu.SemaphoreType.DMA,
            pltpu.SemaphoreType.DMA,
        ],
        compiler_params=pltpu.CompilerParams(use_tc_tiling_on_sc=False),
    )
    def _gather(idx_hbm, table_hbm, out_hbm, idx_v, buf0, buf1, g0, g1, w0, w1):
        bufs = (buf0, buf1)
        gsems = (g0, g1)
        wsems = (w0, w1)
        wid = lax.axis_index("s") * NUM_CORES + lax.axis_index("c")
        c0 = wid * chunks_pw
        pltpu.sync_copy(idx_hbm.at[pl.ds(c0 * CHUNK, chunks_pw * CHUNK)], idx_v)

        def start_gather(g, b):
            pltpu.async_copy(
                table_hbm.at[idx_v.at[pl.ds(g * GROUP, GROUP)]], bufs[b], gsems[b]
            )

        def wait_gather(b):
            pltpu.make_async_copy(
                table_hbm.at[idx_v.at[pl.ds(0, GROUP)]], bufs[b], gsems[b]
            ).wait()

        def start_write(g, b):
            pltpu.async_copy(
                bufs[b], out_hbm.at[pl.ds((c0 + g * KK) * CHUNK, GROUP)], wsems[b]
            )

        def wait_write(b):
            pltpu.make_async_copy(
                bufs[b], out_hbm.at[pl.ds(c0 * CHUNK, GROUP)], wsems[b]
            ).wait()

        start_gather(0, 0)
        wait_gather(0)
        start_write(0, 0)
        start_gather(1, 1)

        @pl.loop(1, ngroups - 1, step=2)
        def _pair(gbase):
            for b in (1, 0):
                g = gbase if b == 1 else gbase + 1
                wait_gather(b)
                start_write(g, b)
                wait_write(1 - b)
                start_gather(g + 1, 1 - b)

        wait_gather(1)
        start_write(ngroups - 1, 1)
        wait_write(0)
        wait_write(1)

    return _gather


@jax.jit
def kernel(token_ids, weights):
    S, T = token_ids.shape
    B = S * T
    D = weights.shape[1]
    idx_flat = token_ids.reshape(B).astype(jnp.int32)
    out = _build(B, D)(idx_flat, weights)
    return out.reshape(S, T, D)


# speedup vs baseline: 1.0103x; 1.0027x over previous
import functools

import jax
import jax.numpy as jnp
from jax import lax
from jax.experimental import pallas as pl
from jax.experimental.pallas import tpu as pltpu
from jax.experimental.pallas import tpu_sc as plsc

NUM_CORES = 2
NUM_SUBCORES = 16
NW = NUM_CORES * NUM_SUBCORES
CHUNK = 128
KK = 2
GROUP = KK * CHUNK
NBUF = 4


@functools.cache
def _build(B, D):
    chunks_pw = B // (NW * CHUNK)
    ngroups = chunks_pw // KK
    assert (ngroups - NBUF) % NBUF == 0 and ngroups >= 2 * NBUF
    mesh = plsc.VectorSubcoreMesh(core_axis_name="c", subcore_axis_name="s")

    @functools.partial(
        pl.kernel,
        out_type=jax.ShapeDtypeStruct((B, D), jnp.float32),
        mesh=mesh,
        scratch_types=[
            pltpu.VMEM((chunks_pw * CHUNK,), jnp.int32),
            pltpu.VMEM((NBUF, GROUP, D), jnp.float32),
            pltpu.SemaphoreType.DMA,
            pltpu.SemaphoreType.DMA,
            pltpu.SemaphoreType.DMA,
            pltpu.SemaphoreType.DMA,
            pltpu.SemaphoreType.DMA,
            pltpu.SemaphoreType.DMA,
            pltpu.SemaphoreType.DMA,
            pltpu.SemaphoreType.DMA,
        ],
        compiler_params=pltpu.CompilerParams(use_tc_tiling_on_sc=False),
    )
    def _gather(idx_hbm, table_hbm, out_hbm, idx_v, bufs, *sems):
        gsems = sems[:NBUF]
        wsems = sems[NBUF:]
        wid = lax.axis_index("s") * NUM_CORES + lax.axis_index("c")
        c0 = wid * chunks_pw
        pltpu.sync_copy(idx_hbm.at[pl.ds(c0 * CHUNK, chunks_pw * CHUNK)], idx_v)

        def start_gather(g, b):
            pltpu.async_copy(
                table_hbm.at[idx_v.at[pl.ds(g * GROUP, GROUP)]],
                bufs.at[b],
                gsems[b],
            )

        def wait_gather(b):
            pltpu.make_async_copy(
                table_hbm.at[idx_v.at[pl.ds(0, GROUP)]], bufs.at[b], gsems[b]
            ).wait()

        def start_write(g, b):
            pltpu.async_copy(
                bufs.at[b], out_hbm.at[pl.ds((c0 + g * KK) * CHUNK, GROUP)], wsems[b]
            )

        def wait_write(b):
            pltpu.make_async_copy(
                bufs.at[b], out_hbm.at[pl.ds(c0 * CHUNK, GROUP)], wsems[b]
            ).wait()

        for b in range(NBUF - 1):
            start_gather(b, b)
        wait_gather(0)
        start_write(0, 0)
        start_gather(NBUF - 1, NBUF - 1)

        @pl.loop(1, ngroups - NBUF + 1, step=NBUF)
        def _span(gbase):
            for j in range(NBUF):
                b = (1 + j) % NBUF
                g = gbase + j
                wait_gather(b)
                start_write(g, b)
                wait_write((b - 1) % NBUF)
                start_gather(g + NBUF - 1, (b - 1) % NBUF)

        for j in range(NBUF - 1):
            g = ngroups - NBUF + 1 + j
            b = g % NBUF
            wait_gather(b)
            start_write(g, b)
            wait_write((b - 1) % NBUF)
        wait_write((ngroups - 1) % NBUF)

    return _gather


@jax.jit
def kernel(token_ids, weights):
    S, T = token_ids.shape
    B = S * T
    D = weights.shape[1]
    idx_flat = token_ids.reshape(B).astype(jnp.int32)
    out = _build(B, D)(idx_flat, weights)
    return out.reshape(S, T, D)


# speedup vs baseline: 1.0510x; 1.0404x over previous
import functools

import jax
import jax.numpy as jnp
from jax import lax
from jax.experimental import pallas as pl
from jax.experimental.pallas import tpu as pltpu
from jax.experimental.pallas import tpu_sc as plsc

NUM_CORES = 2
NUM_SUBCORES = 16
NW = NUM_CORES * NUM_SUBCORES
CHUNK = 128
KK = 1
GROUP = KK * CHUNK
NBUF = 4
DPAD = 128


@functools.cache
def _build(B, D):
    chunks_pw = B // (NW * CHUNK)
    ngroups = chunks_pw // KK
    assert (ngroups - NBUF) % NBUF == 0 and ngroups >= 2 * NBUF
    mesh = plsc.VectorSubcoreMesh(core_axis_name="c", subcore_axis_name="s")

    @functools.partial(
        pl.kernel,
        out_type=jax.ShapeDtypeStruct((B, D), jnp.float32),
        mesh=mesh,
        scratch_types=[
            pltpu.VMEM((chunks_pw * CHUNK,), jnp.int32),
            pltpu.VMEM((NBUF, GROUP, DPAD), jnp.float32),
            pltpu.SemaphoreType.DMA,
            pltpu.SemaphoreType.DMA,
            pltpu.SemaphoreType.DMA,
            pltpu.SemaphoreType.DMA,
            pltpu.SemaphoreType.DMA,
            pltpu.SemaphoreType.DMA,
            pltpu.SemaphoreType.DMA,
            pltpu.SemaphoreType.DMA,
        ],
        compiler_params=pltpu.CompilerParams(use_tc_tiling_on_sc=False),
    )
    def _gather(idx_hbm, table_hbm, out_hbm, idx_v, bufs, *sems):
        gsems = sems[:NBUF]
        wsems = sems[NBUF:]
        wid = lax.axis_index("s") * NUM_CORES + lax.axis_index("c")
        c0 = wid * chunks_pw
        pltpu.sync_copy(idx_hbm.at[pl.ds(c0 * CHUNK, chunks_pw * CHUNK)], idx_v)

        def start_gather(g, b):
            pltpu.async_copy(
                table_hbm.at[idx_v.at[pl.ds(g * GROUP, GROUP)]],
                bufs.at[b],
                gsems[b],
            )

        def wait_gather(b):
            pltpu.make_async_copy(
                table_hbm.at[idx_v.at[pl.ds(0, GROUP)]], bufs.at[b], gsems[b]
            ).wait()

        def start_write(g, b):
            pltpu.async_copy(
                bufs.at[b].at[:, pl.ds(0, D)],
                out_hbm.at[pl.ds((c0 + g * KK) * CHUNK, GROUP)],
                wsems[b],
            )

        def wait_write(b):
            pltpu.make_async_copy(
                bufs.at[b].at[:, pl.ds(0, D)],
                out_hbm.at[pl.ds(c0 * CHUNK, GROUP)],
                wsems[b],
            ).wait()

        for b in range(NBUF - 1):
            start_gather(b, b)
        wait_gather(0)
        start_write(0, 0)
        start_gather(NBUF - 1, NBUF - 1)

        @pl.loop(1, ngroups - NBUF + 1, step=NBUF)
        def _span(gbase):
            for j in range(NBUF):
                b = (1 + j) % NBUF
                g = gbase + j
                wait_gather(b)
                start_write(g, b)
                wait_write((b - 1) % NBUF)
                start_gather(g + NBUF - 1, (b - 1) % NBUF)

        for j in range(NBUF - 1):
            g = ngroups - NBUF + 1 + j
            b = g % NBUF
            wait_gather(b)
            start_write(g, b)
            wait_write((b - 1) % NBUF)
        wait_write((ngroups - 1) % NBUF)

    return _gather


@jax.jit
def kernel(token_ids, weights):
    S, T = token_ids.shape
    B = S * T
    D = weights.shape[1]
    idx_flat = token_ids.reshape(B).astype(jnp.int32)
    wpad = jnp.pad(weights, ((0, 0), (0, DPAD - D)))
    out = _build(B, D)(idx_flat, wpad)
    return out.reshape(S, T, D)
